# Initial kernel scaffold; baseline (speedup 1.0000x reference)
import jax
import jax.numpy as jnp
from jax.experimental import pallas as pl


def kernel(input_ids, kg_enc_input, cross_attn, last_hidden_state, entity_emb, rel_emb, W_mlp, b_mlp, W_lin, W_li, Wq, Wk, Wv, Wo, W_out, Wg, bg, Wc, bc):
    raise NotImplementedError("write your pallas kernel here")


# speedup vs baseline: 1.3342x; 1.3342x over previous
import functools
import math

import jax
import jax.numpy as jnp
from jax import lax
from jax.experimental import pallas as pl
from jax.experimental.pallas import tpu as pltpu
from jax.experimental.pallas import tpu_sc as plsc

B, MAX_LEN, SRC_LEN = 8, 64, 128
N1, N2 = 50, 10
NT = N1 * N2
NTP = 512
VOCAB, REL_VOCAB = 50000, 1000
T_EMBED, HIDDEN, HEADS = 300, 768, 8
DK = HIDDEN // HEADS
LH2 = 2 * HIDDEN
ROWS = B * MAX_LEN

NC, NS, L = 2, 16, 16
NW = NC * NS

GATHER_N = B * NTP
GATHER_PER_W = GATHER_N // NW

ROWS_PER_W = ROWS // NW


def _sc_gather(entity_emb, rel_emb, hidx, ridx, tidx):
    mesh = plsc.VectorSubcoreMesh(
        core_axis_name="c", subcore_axis_name="s",
        num_cores=NC, num_subcores=NS)

    @functools.partial(
        pl.kernel,
        out_type=(
            jax.ShapeDtypeStruct((GATHER_N, T_EMBED), jnp.float32),
            jax.ShapeDtypeStruct((GATHER_N, T_EMBED), jnp.float32),
            jax.ShapeDtypeStruct((GATHER_N, T_EMBED), jnp.float32),
        ),
        mesh=mesh,
        scratch_types=[
            pltpu.VMEM((GATHER_PER_W,), jnp.int32),
            pltpu.VMEM((GATHER_PER_W, T_EMBED), jnp.float32),
            pltpu.SemaphoreType.DMA,
        ],
        compiler_params=pltpu.CompilerParams(use_tc_tiling_on_sc=False),
    )
    def k(ent_hbm, rel_hbm, hi_hbm, ri_hbm, ti_hbm,
          ho_hbm, ro_hbm, to_hbm, idx_v, rows_v, sem):
        wid = lax.axis_index("s") * NC + lax.axis_index("c")
        base = wid * GATHER_PER_W
        for (i_hbm, tbl, o_hbm) in ((hi_hbm, ent_hbm, ho_hbm),
                                    (ri_hbm, rel_hbm, ro_hbm),
                                    (ti_hbm, ent_hbm, to_hbm)):
            pltpu.sync_copy(i_hbm.at[pl.ds(base, GATHER_PER_W)], idx_v)
            pltpu.async_copy(tbl.at[idx_v], rows_v, sem).wait()
            pltpu.sync_copy(rows_v, o_hbm.at[pl.ds(base, GATHER_PER_W)])

    return k(entity_emb, rel_emb, hidx, ridx, tidx)


def _small_kernel(h_ref, r_ref, t_ref, wmlp_ref, bmlp_ref, wlin_ref,
                  lhs_ref, wli_ref, wq_ref, wk_ref, wv_ref, wo_ref, wc_ref,
                  bc_ref, cross_ref,
                  outh_ref, attn_ref, dlg_ref, pcon_ref):
    f32 = jnp.float32
    h = h_ref[0]
    r = r_ref[0]
    t = t_ref[0]
    wmlp = wmlp_ref[...]
    t3 = (jnp.dot(h, wmlp[0:T_EMBED], preferred_element_type=f32)
          + jnp.dot(r, wmlp[T_EMBED:2 * T_EMBED], preferred_element_type=f32)
          + jnp.dot(t, wmlp[2 * T_EMBED:3 * T_EMBED], preferred_element_type=f32)
          + bmlp_ref[...])
    tl = jnp.dot(t3, wlin_ref[...], preferred_element_type=f32)
    kk = jnp.dot(tl, wk_ref[...], preferred_element_type=f32)
    vv = jnp.dot(tl, wv_ref[...], preferred_element_type=f32)
    oh = jnp.dot(lhs_ref[0], wli_ref[...], preferred_element_type=f32)
    q = jnp.dot(oh, wq_ref[...], preferred_element_type=f32)

    key_mask = lax.broadcasted_iota(jnp.int32, (1, NTP), 1) < NT
    inv_sqrt_dk = 1.0 / math.sqrt(DK)
    attn_acc = jnp.zeros((MAX_LEN, NTP), f32)
    ctx_parts = []
    for hh in range(HEADS):
        sl = slice(hh * DK, (hh + 1) * DK)
        s = lax.dot_general(q[:, sl], kk[:, sl],
                            (((1,), (1,)), ((), ())),
                            preferred_element_type=f32) * inv_sqrt_dk
        s = jnp.where(key_mask, s, -1e30)
        s = s - jnp.max(s, axis=-1, keepdims=True)
        e = jnp.exp(s)
        p = e / jnp.sum(e, axis=-1, keepdims=True)
        attn_acc = attn_acc + p
        ctx_parts.append(jnp.dot(p, vv[:, sl], preferred_element_type=f32))
    ctx = jnp.concatenate(ctx_parts, axis=1)
    woc = jnp.dot(wo_ref[...], wc_ref[...], preferred_element_type=f32)
    pcon = jax.nn.sigmoid(jnp.dot(ctx, woc, preferred_element_type=f32)
                          + bc_ref[0, 0])

    outh_ref[0] = oh
    attn_ref[0] = attn_acc * (1.0 / HEADS)
    dlg_ref[0] = jnp.mean(cross_ref[0], axis=0)
    pcon_ref[0] = pcon


def _small_stages(h_rows, r_rows, t_rows, W_mlp, b_mlp, W_lin,
                  last_hidden_state, W_li, Wq, Wk, Wv, Wo, Wc, bc, cross_attn):
    f32 = jnp.float32
    full = lambda shape: pl.BlockSpec(shape, lambda j: (0,) * len(shape))
    batch = lambda shape: pl.BlockSpec((1,) + shape,
                                       lambda j: (j,) + (0,) * len(shape))
    return pl.pallas_call(
        _small_kernel,
        grid=(B,),
        in_specs=[
            batch((NTP, T_EMBED)), batch((NTP, T_EMBED)), batch((NTP, T_EMBED)),
            full((3 * T_EMBED, 3 * T_EMBED)), full((1, 3 * T_EMBED)),
            full((3 * T_EMBED, HIDDEN)),
            batch((MAX_LEN, LH2)), full((LH2, HIDDEN)),
            full((HIDDEN, HIDDEN)), full((HIDDEN, HIDDEN)),
            full((HIDDEN, HIDDEN)), full((HIDDEN, HIDDEN)),
            full((HIDDEN, 1)), full((1, 1)),
            batch((12, MAX_LEN, SRC_LEN)),
        ],
        out_specs=[
            batch((MAX_LEN, HIDDEN)), batch((MAX_LEN, NTP)),
            batch((MAX_LEN, SRC_LEN)), batch((MAX_LEN, 1)),
        ],
        out_shape=[
            jax.ShapeDtypeStruct((B, MAX_LEN, HIDDEN), f32),
            jax.ShapeDtypeStruct((B, MAX_LEN, NTP), f32),
            jax.ShapeDtypeStruct((B, MAX_LEN, SRC_LEN), f32),
            jax.ShapeDtypeStruct((B, MAX_LEN, 1), f32),
        ],
        compiler_params=pltpu.CompilerParams(
            dimension_semantics=("arbitrary",)),
    )(h_rows.reshape(B, NTP, T_EMBED), r_rows.reshape(B, NTP, T_EMBED),
      t_rows.reshape(B, NTP, T_EMBED), W_mlp, b_mlp.reshape(1, -1), W_lin,
      last_hidden_state, W_li, Wq, Wk, Wv, Wo, Wc, bc.reshape(1, 1),
      cross_attn)


VTILE = 1024
NVT = (VOCAB + VTILE - 1) // VTILE


def _vocab_kernel(outh_ref, wout_ref, wg_ref, logits_ref, g_ref):
    j = pl.program_id(0)
    a = outh_ref[...].astype(jnp.bfloat16)
    w = wout_ref[...].astype(jnp.bfloat16)
    acc = jnp.dot(a, w, preferred_element_type=jnp.float32)
    logits_ref[...] = acc
    rem = VOCAB - j * VTILE
    col_ok = lax.broadcasted_iota(jnp.int32, (ROWS, VTILE), 1) < rem
    accm = jnp.where(col_ok, acc, 0.0)
    wg = jnp.where(lax.broadcasted_iota(jnp.int32, (VTILE, 1), 0) < rem,
                   wg_ref[...], 0.0)
    gpart = jnp.dot(accm, wg, preferred_element_type=jnp.float32)

    @pl.when(j == 0)
    def _():
        g_ref[...] = jnp.zeros_like(g_ref)

    g_ref[...] += gpart


def _vocab_pass(out_h, W_out, Wg):
    f32 = jnp.float32
    return pl.pallas_call(
        _vocab_kernel,
        grid=(NVT,),
        in_specs=[
            pl.BlockSpec((ROWS, HIDDEN), lambda j: (0, 0)),
            pl.BlockSpec((HIDDEN, VTILE), lambda j: (0, j)),
            pl.BlockSpec((VTILE, 1), lambda j: (j, 0)),
        ],
        out_specs=[
            pl.BlockSpec((ROWS, VTILE), lambda j: (0, j)),
            pl.BlockSpec((ROWS, 1), lambda j: (0, 0)),
        ],
        out_shape=[
            jax.ShapeDtypeStruct((ROWS, VOCAB), f32),
            jax.ShapeDtypeStruct((ROWS, 1), f32),
        ],
        compiler_params=pltpu.CompilerParams(
            dimension_semantics=("arbitrary",)),
    )(out_h, W_out, Wg)


def _sc_mix(logits_flat, ids_flat, tail_flat, dlg_flat, attn_flat,
            c1_flat, c0_flat, ck_flat):
    mesh = plsc.VectorSubcoreMesh(
        core_axis_name="c", subcore_axis_name="s",
        num_cores=NC, num_subcores=NS)

    @functools.partial(
        pl.kernel,
        out_type=jax.ShapeDtypeStruct((ROWS * VOCAB,), jnp.float32),
        mesh=mesh,
        scratch_types=[
            pltpu.VMEM((VOCAB,), jnp.float32),
            pltpu.VMEM((SRC_LEN,), jnp.int32),
            pltpu.VMEM((NTP,), jnp.int32),
            pltpu.VMEM((SRC_LEN,), jnp.float32),
            pltpu.VMEM((NTP,), jnp.float32),
            pltpu.VMEM((L,), jnp.float32),
            pltpu.VMEM((L,), jnp.float32),
            pltpu.VMEM((L,), jnp.float32),
        ],
        compiler_params=pltpu.CompilerParams(needs_layout_passes=False),
    )
    def k(log_hbm, ids_hbm, tail_hbm, dlg_hbm, attn_hbm,
          c1_hbm, c0_hbm, ck_hbm, out_hbm,
          rowbuf, idsbuf, tailbuf, dlgbuf, attnbuf, c1buf, c0buf, ckbuf):
        wid = lax.axis_index("s") * NC + lax.axis_index("c")
        b = wid // (NW // B)
        pltpu.sync_copy(ids_hbm.at[pl.ds(b * SRC_LEN, SRC_LEN)], idsbuf)
        pltpu.sync_copy(tail_hbm.at[pl.ds(b * NTP, NTP)], tailbuf)

        def row_body(i, _):
            r = wid * ROWS_PER_W + i
            roff = pl.multiple_of(r * VOCAB, 8)
            pltpu.sync_copy(log_hbm.at[pl.ds(roff, VOCAB)], rowbuf)
            pltpu.sync_copy(dlg_hbm.at[pl.ds(pl.multiple_of(r * SRC_LEN, 8),
                                             SRC_LEN)], dlgbuf)
            pltpu.sync_copy(attn_hbm.at[pl.ds(pl.multiple_of(r * NTP, 8),
                                              NTP)], attnbuf)
            pltpu.sync_copy(c1_hbm.at[pl.ds(pl.multiple_of(r * L, 8), L)],
                            c1buf)
            pltpu.sync_copy(c0_hbm.at[pl.ds(pl.multiple_of(r * L, 8), L)],
                            c0buf)
            pltpu.sync_copy(ck_hbm.at[pl.ds(pl.multiple_of(r * L, 8), L)],
                            ckbuf)
            c1 = c1buf[...]
            c0 = c0buf[...]
            ck = ckbuf[...]

            @plsc.parallel_loop(0, VOCAB // L, unroll=5)
            def _(jj):
                sl = pl.ds(jj * L, L)
                rowbuf[sl] = rowbuf[sl] * c1

            for j in range(SRC_LEN // L):
                sl = pl.ds(j * L, L)
                plsc.addupdate_scatter(rowbuf, [idsbuf[sl]], dlgbuf[sl] * c0)
            for j in range(NTP // L):
                sl = pl.ds(j * L, L)
                plsc.addupdate_scatter(rowbuf, [tailbuf[sl]], attnbuf[sl] * ck)

            pltpu.sync_copy(rowbuf, out_hbm.at[pl.ds(roff, VOCAB)])
            return ()

        lax.fori_loop(0, ROWS_PER_W, row_body, (), unroll=False)

    return k(logits_flat, ids_flat, tail_flat, dlg_flat, attn_flat,
             c1_flat, c0_flat, ck_flat)


def kernel(input_ids, kg_enc_input, cross_attn, last_hidden_state, entity_emb,
           rel_emb, W_mlp, b_mlp, W_lin, W_li, Wq, Wk, Wv, Wo, W_out, Wg, bg,
           Wc, bc):
    f32 = jnp.float32
    head = kg_enc_input[..., 0].reshape(B, NT)
    rel = kg_enc_input[..., 1].reshape(B, NT)
    tail = kg_enc_input[..., 2].reshape(B, NT)
    pad = ((0, 0), (0, NTP - NT))
    hidx = jnp.pad(head, pad).reshape(-1)
    ridx = jnp.pad(rel, pad).reshape(-1)
    tidx = jnp.pad(tail, pad).reshape(-1)

    h_rows, r_rows, t_rows = _sc_gather(entity_emb, rel_emb, hidx, ridx, tidx)

    out_h, attn, dlg, pcon = _small_stages(
        h_rows, r_rows, t_rows, W_mlp, b_mlp, W_lin,
        last_hidden_state, W_li, Wq, Wk, Wv, Wo, Wc, bc, cross_attn)

    logits, g = _vocab_pass(out_h.reshape(ROWS, HIDDEN), W_out, Wg)

    p_gen = jax.nn.sigmoid(g.reshape(ROWS) + bg[0])
    pc = pcon.reshape(ROWS)
    c1 = (1.0 - pc) * p_gen
    c0 = (1.0 - pc) * (1.0 - p_gen)
    ck = pc
    splat = lambda x: jnp.broadcast_to(x[:, None], (ROWS, L)).reshape(-1)

    out = _sc_mix(
        logits.reshape(-1),
        input_ids.reshape(-1).astype(jnp.int32),
        tidx,
        dlg.reshape(-1).astype(f32),
        attn.reshape(-1).astype(f32),
        splat(c1), splat(c0), splat(ck))

    return out.reshape(B, MAX_LEN, VOCAB)


# speedup vs baseline: 1.7248x; 1.2928x over previous
import functools
import math

import jax
import jax.numpy as jnp
from jax import lax
from jax.experimental import pallas as pl
from jax.experimental.pallas import tpu as pltpu
from jax.experimental.pallas import tpu_sc as plsc

B, MAX_LEN, SRC_LEN = 8, 64, 128
N1, N2 = 50, 10
NT = N1 * N2
NTP = 512
VOCAB, REL_VOCAB = 50000, 1000
T_EMBED, HIDDEN, HEADS = 300, 768, 8
DK = HIDDEN // HEADS
LH2 = 2 * HIDDEN
ROWS = B * MAX_LEN

NC, NS, L = 2, 16, 16
NW = NC * NS

GATHER_N = B * NTP
GATHER_PER_W = GATHER_N // NW

ROWS_PER_W = ROWS // NW


def _sc_gather(entity_emb, rel_emb, hidx, ridx, tidx):
    mesh = plsc.VectorSubcoreMesh(
        core_axis_name="c", subcore_axis_name="s",
        num_cores=NC, num_subcores=NS)

    @functools.partial(
        pl.kernel,
        out_type=(
            jax.ShapeDtypeStruct((GATHER_N, T_EMBED), jnp.float32),
            jax.ShapeDtypeStruct((GATHER_N, T_EMBED), jnp.float32),
            jax.ShapeDtypeStruct((GATHER_N, T_EMBED), jnp.float32),
        ),
        mesh=mesh,
        scratch_types=[
            pltpu.VMEM((GATHER_PER_W,), jnp.int32),
            pltpu.VMEM((GATHER_PER_W, T_EMBED), jnp.float32),
            pltpu.SemaphoreType.DMA,
        ],
        compiler_params=pltpu.CompilerParams(use_tc_tiling_on_sc=False),
    )
    def k(ent_hbm, rel_hbm, hi_hbm, ri_hbm, ti_hbm,
          ho_hbm, ro_hbm, to_hbm, idx_v, rows_v, sem):
        wid = lax.axis_index("s") * NC + lax.axis_index("c")
        base = wid * GATHER_PER_W
        for (i_hbm, tbl, o_hbm) in ((hi_hbm, ent_hbm, ho_hbm),
                                    (ri_hbm, rel_hbm, ro_hbm),
                                    (ti_hbm, ent_hbm, to_hbm)):
            pltpu.sync_copy(i_hbm.at[pl.ds(base, GATHER_PER_W)], idx_v)
            pltpu.async_copy(tbl.at[idx_v], rows_v, sem).wait()
            pltpu.sync_copy(rows_v, o_hbm.at[pl.ds(base, GATHER_PER_W)])

    return k(entity_emb, rel_emb, hidx, ridx, tidx)


def _small_kernel(h_ref, r_ref, t_ref, wmlp_ref, bmlp_ref, wlin_ref,
                  lhs_ref, wli_ref, wq_ref, wk_ref, wv_ref, wo_ref, wc_ref,
                  bc_ref, cross_ref,
                  outh_ref, attn_ref, dlg_ref, pcon_ref):
    f32 = jnp.float32
    h = h_ref[0]
    r = r_ref[0]
    t = t_ref[0]
    wmlp = wmlp_ref[...]
    t3 = (jnp.dot(h, wmlp[0:T_EMBED], preferred_element_type=f32)
          + jnp.dot(r, wmlp[T_EMBED:2 * T_EMBED], preferred_element_type=f32)
          + jnp.dot(t, wmlp[2 * T_EMBED:3 * T_EMBED], preferred_element_type=f32)
          + bmlp_ref[...])
    tl = jnp.dot(t3, wlin_ref[...], preferred_element_type=f32)
    kk = jnp.dot(tl, wk_ref[...], preferred_element_type=f32)
    vv = jnp.dot(tl, wv_ref[...], preferred_element_type=f32)
    oh = jnp.dot(lhs_ref[0], wli_ref[...], preferred_element_type=f32)
    q = jnp.dot(oh, wq_ref[...], preferred_element_type=f32)

    key_mask = lax.broadcasted_iota(jnp.int32, (1, NTP), 1) < NT
    inv_sqrt_dk = 1.0 / math.sqrt(DK)
    attn_acc = jnp.zeros((MAX_LEN, NTP), f32)
    ctx_parts = []
    for hh in range(HEADS):
        sl = slice(hh * DK, (hh + 1) * DK)
        s = lax.dot_general(q[:, sl], kk[:, sl],
                            (((1,), (1,)), ((), ())),
                            preferred_element_type=f32) * inv_sqrt_dk
        s = jnp.where(key_mask, s, -1e30)
        s = s - jnp.max(s, axis=-1, keepdims=True)
        e = jnp.exp(s)
        p = e / jnp.sum(e, axis=-1, keepdims=True)
        attn_acc = attn_acc + p
        ctx_parts.append(jnp.dot(p, vv[:, sl], preferred_element_type=f32))
    ctx = jnp.concatenate(ctx_parts, axis=1)
    woc = jnp.dot(wo_ref[...], wc_ref[...], preferred_element_type=f32)
    pcon = jax.nn.sigmoid(jnp.dot(ctx, woc, preferred_element_type=f32)
                          + bc_ref[0, 0])

    outh_ref[0] = oh
    attn_ref[0] = attn_acc * (1.0 / HEADS)
    dlg_ref[0] = jnp.mean(cross_ref[0], axis=0)
    pcon_ref[0] = pcon


def _small_stages(h_rows, r_rows, t_rows, W_mlp, b_mlp, W_lin,
                  last_hidden_state, W_li, Wq, Wk, Wv, Wo, Wc, bc, cross_attn):
    f32 = jnp.float32
    full = lambda shape: pl.BlockSpec(shape, lambda j: (0,) * len(shape))
    batch = lambda shape: pl.BlockSpec((1,) + shape,
                                       lambda j: (j,) + (0,) * len(shape))
    return pl.pallas_call(
        _small_kernel,
        grid=(B,),
        in_specs=[
            batch((NTP, T_EMBED)), batch((NTP, T_EMBED)), batch((NTP, T_EMBED)),
            full((3 * T_EMBED, 3 * T_EMBED)), full((1, 3 * T_EMBED)),
            full((3 * T_EMBED, HIDDEN)),
            batch((MAX_LEN, LH2)), full((LH2, HIDDEN)),
            full((HIDDEN, HIDDEN)), full((HIDDEN, HIDDEN)),
            full((HIDDEN, HIDDEN)), full((HIDDEN, HIDDEN)),
            full((HIDDEN, 1)), full((1, 1)),
            batch((12, MAX_LEN, SRC_LEN)),
        ],
        out_specs=[
            batch((MAX_LEN, HIDDEN)), batch((MAX_LEN, NTP)),
            batch((MAX_LEN, SRC_LEN)), batch((MAX_LEN, 1)),
        ],
        out_shape=[
            jax.ShapeDtypeStruct((B, MAX_LEN, HIDDEN), f32),
            jax.ShapeDtypeStruct((B, MAX_LEN, NTP), f32),
            jax.ShapeDtypeStruct((B, MAX_LEN, SRC_LEN), f32),
            jax.ShapeDtypeStruct((B, MAX_LEN, 1), f32),
        ],
        compiler_params=pltpu.CompilerParams(
            dimension_semantics=("arbitrary",)),
    )(h_rows.reshape(B, NTP, T_EMBED), r_rows.reshape(B, NTP, T_EMBED),
      t_rows.reshape(B, NTP, T_EMBED), W_mlp, b_mlp.reshape(1, -1), W_lin,
      last_hidden_state, W_li, Wq, Wk, Wv, Wo, Wc, bc.reshape(1, 1),
      cross_attn)


VTILE = 1024
NVT = (VOCAB + VTILE - 1) // VTILE


def _vocab_kernel(outh_ref, wout_ref, wg_ref, logits_ref, g_ref):
    j = pl.program_id(0)
    a = outh_ref[...].astype(jnp.bfloat16)
    w = wout_ref[...].astype(jnp.bfloat16)
    acc = jnp.dot(a, w, preferred_element_type=jnp.float32)
    logits_ref[...] = acc
    rem = VOCAB - j * VTILE
    col_ok = lax.broadcasted_iota(jnp.int32, (ROWS, VTILE), 1) < rem
    accm = jnp.where(col_ok, acc, 0.0)
    wg = jnp.where(lax.broadcasted_iota(jnp.int32, (VTILE, 1), 0) < rem,
                   wg_ref[...], 0.0)
    gpart = jnp.dot(accm, wg, preferred_element_type=jnp.float32)

    @pl.when(j == 0)
    def _():
        g_ref[...] = jnp.zeros_like(g_ref)

    g_ref[...] += gpart


def _vocab_pass(out_h, W_out, Wg):
    f32 = jnp.float32
    return pl.pallas_call(
        _vocab_kernel,
        grid=(NVT,),
        in_specs=[
            pl.BlockSpec((ROWS, HIDDEN), lambda j: (0, 0)),
            pl.BlockSpec((HIDDEN, VTILE), lambda j: (0, j)),
            pl.BlockSpec((VTILE, 1), lambda j: (j, 0)),
        ],
        out_specs=[
            pl.BlockSpec((ROWS, VTILE), lambda j: (0, j)),
            pl.BlockSpec((ROWS, 1), lambda j: (0, 0)),
        ],
        out_shape=[
            jax.ShapeDtypeStruct((ROWS, VOCAB), f32),
            jax.ShapeDtypeStruct((ROWS, 1), f32),
        ],
        compiler_params=pltpu.CompilerParams(
            dimension_semantics=("arbitrary",)),
    )(out_h, W_out, Wg)


def _sc_mix(logits, ids, tail, dlg, attn, c1, c0, ck):
    mesh = plsc.VectorSubcoreMesh(
        core_axis_name="c", subcore_axis_name="s",
        num_cores=NC, num_subcores=NS)

    @functools.partial(
        pl.kernel,
        out_type=jax.ShapeDtypeStruct((ROWS, VOCAB), jnp.float32),
        mesh=mesh,
        scratch_types=[
            pltpu.VMEM((VOCAB,), jnp.float32),
            pltpu.VMEM((SRC_LEN,), jnp.int32),
            pltpu.VMEM((NTP,), jnp.int32),
            pltpu.VMEM((SRC_LEN,), jnp.float32),
            pltpu.VMEM((NTP,), jnp.float32),
            pltpu.VMEM((L,), jnp.float32),
            pltpu.VMEM((L,), jnp.float32),
            pltpu.VMEM((L,), jnp.float32),
        ],
        compiler_params=pltpu.CompilerParams(needs_layout_passes=False),
    )
    def k(log_hbm, ids_hbm, tail_hbm, dlg_hbm, attn_hbm,
          c1_hbm, c0_hbm, ck_hbm, out_hbm,
          rowbuf, idsbuf, tailbuf, dlgbuf, attnbuf, c1buf, c0buf, ckbuf):
        wid = lax.axis_index("s") * NC + lax.axis_index("c")
        b = wid // (NW // B)
        pltpu.sync_copy(ids_hbm.at[b], idsbuf)
        pltpu.sync_copy(tail_hbm.at[b], tailbuf)

        def row_body(i, _):
            r = wid * ROWS_PER_W + i
            pltpu.sync_copy(log_hbm.at[r], rowbuf)
            pltpu.sync_copy(dlg_hbm.at[r], dlgbuf)
            pltpu.sync_copy(attn_hbm.at[r], attnbuf)
            pltpu.sync_copy(c1_hbm.at[r], c1buf)
            pltpu.sync_copy(c0_hbm.at[r], c0buf)
            pltpu.sync_copy(ck_hbm.at[r], ckbuf)
            c1 = c1buf[...]
            c0 = c0buf[...]
            ck = ckbuf[...]

            @plsc.parallel_loop(0, VOCAB // L, unroll=5)
            def _(jj):
                sl = pl.ds(jj * L, L)
                rowbuf[sl] = rowbuf[sl] * c1

            for j in range(SRC_LEN // L):
                sl = pl.ds(j * L, L)
                plsc.addupdate_scatter(rowbuf, [idsbuf[sl]], dlgbuf[sl] * c0)
            for j in range(NTP // L):
                sl = pl.ds(j * L, L)
                plsc.addupdate_scatter(rowbuf, [tailbuf[sl]], attnbuf[sl] * ck)

            pltpu.sync_copy(rowbuf, out_hbm.at[r])
            return ()

        lax.fori_loop(0, ROWS_PER_W, row_body, (), unroll=False)

    return k(logits, ids, tail, dlg, attn, c1, c0, ck)


def kernel(input_ids, kg_enc_input, cross_attn, last_hidden_state, entity_emb,
           rel_emb, W_mlp, b_mlp, W_lin, W_li, Wq, Wk, Wv, Wo, W_out, Wg, bg,
           Wc, bc):
    f32 = jnp.float32
    head = kg_enc_input[..., 0].reshape(B, NT)
    rel = kg_enc_input[..., 1].reshape(B, NT)
    tail = kg_enc_input[..., 2].reshape(B, NT)
    pad = ((0, 0), (0, NTP - NT))
    hidx = jnp.pad(head, pad).reshape(-1)
    ridx = jnp.pad(rel, pad).reshape(-1)
    tidx = jnp.pad(tail, pad).reshape(-1)

    h_rows, r_rows, t_rows = _sc_gather(entity_emb, rel_emb, hidx, ridx, tidx)

    out_h, attn, dlg, pcon = _small_stages(
        h_rows, r_rows, t_rows, W_mlp, b_mlp, W_lin,
        last_hidden_state, W_li, Wq, Wk, Wv, Wo, Wc, bc, cross_attn)

    logits, g = _vocab_pass(out_h.reshape(ROWS, HIDDEN), W_out, Wg)

    p_gen = jax.nn.sigmoid(g.reshape(ROWS) + bg[0])
    pc = pcon.reshape(ROWS)
    c1 = (1.0 - pc) * p_gen
    c0 = (1.0 - pc) * (1.0 - p_gen)
    ck = pc
    splat = lambda x: jnp.broadcast_to(x[:, None], (ROWS, L))

    out = _sc_mix(
        logits,
        input_ids.astype(jnp.int32),
        tidx.reshape(B, NTP),
        dlg.reshape(ROWS, SRC_LEN),
        attn.reshape(ROWS, NTP),
        splat(c1), splat(c0), splat(ck))

    return out.reshape(B, MAX_LEN, VOCAB)
